# baseline (speedup 1.0000x reference)
import functools

import jax
import jax.numpy as jnp
from jax import lax
from jax.experimental import pallas as pl
from jax.experimental.pallas import tpu as pltpu
from jax.experimental.pallas import tpu_sc as plsc

N = 10000
E = 320000
F_IN = 128
H = 64
LH = 32
NT = 10

N_TAB = 10112
NC = 2
NS = 16
NW = NC * NS
CH = 128
CPW = 80
E_PAD = NW * CPW * CH
NPT = N_TAB // NS
WD = 16

_f32 = jnp.float32

_mesh = plsc.VectorSubcoreMesh(core_axis_name="c", subcore_axis_name="s")
_sc_params = pltpu.CompilerParams(use_tc_tiling_on_sc=False)


@functools.partial(
    pl.kernel,
    out_type=jax.ShapeDtypeStruct((NC, N_TAB, WD), _f32),
    mesh=_mesh,
    scratch_types=[
        pltpu.VMEM((CPW, CH), jnp.int32),
        pltpu.VMEM((CH, WD), _f32),
        pltpu.VMEM_SHARED((N_TAB, WD), _f32),
        pltpu.SemaphoreType.DMA,
    ],
    compiler_params=_sc_params,
)
def _deg_kernel(dst_hbm, ones_hbm, zeros_hbm, out_hbm, dst_v, ones_v,
                acc_sh, sem):
    cid = lax.axis_index("c")
    sid = lax.axis_index("s")
    wid = sid * NC + cid
    pltpu.sync_copy(zeros_hbm, acc_sh.at[pl.ds(sid * NPT, NPT)])
    pltpu.sync_copy(ones_hbm, ones_v)
    pltpu.sync_copy(dst_hbm.at[wid], dst_v)
    plsc.subcore_barrier()

    def body(j, carry):
        pltpu.async_copy(ones_v, acc_sh.at[dst_v.at[j]], sem, add=True)

        @pl.when(j >= 8)
        def _():
            pltpu.make_async_copy(ones_v, acc_sh.at[dst_v.at[j - 8]],
                                  sem).wait()

        return carry

    lax.fori_loop(0, CPW, body, 0)
    for k in range(8):
        pltpu.make_async_copy(ones_v, acc_sh.at[dst_v.at[CPW - 8 + k]],
                              sem).wait()
    plsc.subcore_barrier()
    pltpu.sync_copy(acc_sh.at[pl.ds(sid * NPT, NPT)],
                    out_hbm.at[cid, pl.ds(sid * NPT, NPT)])


@functools.partial(
    pl.kernel,
    out_type=jax.ShapeDtypeStruct((NC, N_TAB, H), _f32),
    mesh=_mesh,
    scratch_types=[
        pltpu.VMEM((CPW, CH), jnp.int32),
        pltpu.VMEM((CPW, CH), jnp.int32),
        [pltpu.VMEM((CH, H), _f32)] * 8,
        pltpu.VMEM_SHARED((N_TAB, H), _f32),
        [pltpu.SemaphoreType.DMA] * 8,
        [pltpu.SemaphoreType.DMA] * 8,
    ],
    compiler_params=_sc_params,
)
def _scatter_kernel(g_hbm, src_hbm, dst_hbm, zeros_hbm, out_hbm, src_v, dst_v,
                    rows, acc_sh, gsems, ssems):
    cid = lax.axis_index("c")
    sid = lax.axis_index("s")
    wid = sid * NC + cid
    pltpu.sync_copy(zeros_hbm, acc_sh.at[pl.ds(sid * NPT, NPT)])
    pltpu.sync_copy(src_hbm.at[wid], src_v)
    pltpu.sync_copy(dst_hbm.at[wid], dst_v)
    plsc.subcore_barrier()

    NB = 8
    LAG = NB // 2
    for b in range(NB):
        pltpu.async_copy(g_hbm.at[src_v.at[b]], rows[b], gsems[b])

    def body(i, carry):
        j0 = NB * i
        for b in range(NB):
            j = j0 + b
            pltpu.make_async_copy(g_hbm.at[src_v.at[j]], rows[b],
                                  gsems[b]).wait()
            pltpu.async_copy(rows[b], acc_sh.at[dst_v.at[j]], ssems[b],
                             add=True)
            jr = j + LAG
            br = (b + LAG) % NB

            @pl.when(jnp.logical_and(jr >= NB, jr < CPW))
            def _():
                pltpu.make_async_copy(rows[br], acc_sh.at[dst_v.at[jr - NB]],
                                      ssems[br]).wait()
                pltpu.async_copy(g_hbm.at[src_v.at[jr]], rows[br], gsems[br])

        return carry

    lax.fori_loop(0, CPW // NB, body, 0)
    for b in range(NB):
        pltpu.make_async_copy(rows[b], acc_sh.at[dst_v.at[CPW - NB + b]],
                              ssems[b]).wait()
    plsc.subcore_barrier()
    pltpu.sync_copy(acc_sh.at[pl.ds(sid * NPT, NPT)],
                    out_hbm.at[cid, pl.ds(sid * NPT, NPT)])


def _prep_body(degacc, x, w1, dinv_o, g1_o):
    deg = degacc[0] + degacc[1]
    deg0 = deg[:, 0:1] + 1.0
    row = lax.broadcasted_iota(jnp.int32, (N_TAB, 1), 0)
    dinv = jnp.where(row < N, lax.rsqrt(deg0), 0.0)
    dinv_o[...] = dinv
    g1_o[...] = dinv * jnp.dot(x[...], w1[...], preferred_element_type=_f32)


_prep_call = pl.pallas_call(
    _prep_body,
    out_shape=(jax.ShapeDtypeStruct((N_TAB, 1), _f32),
               jax.ShapeDtypeStruct((N_TAB, H), _f32)),
)


def _mid_body(acc, g, dinv, b, w, gout):
    s = acc[0] + acc[1] + g[...]
    h = jnp.maximum(dinv[...] * s + b[...], 0.0)
    gout[...] = dinv[...] * jnp.dot(h, w[...], preferred_element_type=_f32)


_mid_call = pl.pallas_call(
    _mid_body,
    out_shape=jax.ShapeDtypeStruct((N_TAB, H), _f32),
)


def _epi_body(acc, g3, dinv, b3, wi0, wg0, wo0, bi0, bg0, bo0,
              wi1, wg1, wo1, bi1, bg1, bo1,
              vw1, vb1, vw2, vb2, tw1, tb1, tw2, tb2, vol_o, typ_o):
    s = acc[0] + acc[1] + g3[...]
    h = jnp.maximum(dinv[...] * s + b3[...], 0.0)

    def cell(hh, wi, wg, wo, bi, bg, bo):
        ig = jax.nn.sigmoid(jnp.dot(hh, wi[...], preferred_element_type=_f32)
                            + bi[...])
        gg = jnp.tanh(jnp.dot(hh, wg[...], preferred_element_type=_f32)
                      + bg[...])
        og = jax.nn.sigmoid(jnp.dot(hh, wo[...], preferred_element_type=_f32)
                            + bo[...])
        return og * jnp.tanh(ig * gg)

    hl = cell(h, wi0, wg0, wo0, bi0, bg0, bo0)
    hl = cell(hl, wi1, wg1, wo1, bi1, bg1, bo1)
    v = jnp.maximum(jnp.dot(hl, vw1[...], preferred_element_type=_f32)
                    + vb1[...], 0.0)
    vol_o[...] = jnp.dot(v, vw2[...], preferred_element_type=_f32) + vb2[...]
    t = jnp.maximum(jnp.dot(hl, tw1[...], preferred_element_type=_f32)
                    + tb1[...], 0.0)
    typ_o[...] = jnp.dot(t, tw2[...], preferred_element_type=_f32) + tb2[...]


_epi_call = pl.pallas_call(
    _epi_body,
    out_shape=(jax.ShapeDtypeStruct((N_TAB, 1), _f32),
               jax.ShapeDtypeStruct((N_TAB, NT), _f32)),
)


@jax.jit
def _run(x, edge_index, W1, b1, W2, b2, W3, b3, Wih0, bih0, bhh0,
         Wih1, bih1, bhh1, vW1, vb1, vW2, vb2, tW1, tb1, tW2, tb2):
    pad = E_PAD - E
    padrows = N + (jnp.arange(pad, dtype=jnp.int32) % (N_TAB - N))
    src_p = jnp.concatenate([edge_index[0], padrows]).reshape(NW, CPW, CH)
    dst_p = jnp.concatenate([edge_index[1], padrows]).reshape(NW, CPW, CH)
    x_pad = jnp.pad(x, ((0, N_TAB - N), (0, 0)))

    zeros16 = jnp.zeros((NPT, WD), _f32)
    ones16 = jnp.ones((CH, WD), _f32)
    zeros64 = jnp.zeros((NPT, H), _f32)

    degacc = _deg_kernel(dst_p, ones16, zeros16)
    dinv, g1 = _prep_call(degacc, x_pad, W1)
    acc1 = _scatter_kernel(g1, src_p, dst_p, zeros64)
    g2 = _mid_call(acc1, g1, dinv, b1.reshape(1, H), W2)
    acc2 = _scatter_kernel(g2, src_p, dst_p, zeros64)
    g3 = _mid_call(acc2, g2, dinv, b2.reshape(1, H), W3)
    acc3 = _scatter_kernel(g3, src_p, dst_p, zeros64)

    bc0 = bih0 + bhh0
    bc1 = bih1 + bhh1
    wi0, wg0, wo0 = Wih0[:LH].T, Wih0[2 * LH:3 * LH].T, Wih0[3 * LH:].T
    wi1, wg1, wo1 = Wih1[:LH].T, Wih1[2 * LH:3 * LH].T, Wih1[3 * LH:].T
    vol, typ = _epi_call(
        acc3, g3, dinv, b3.reshape(1, H),
        wi0, wg0, wo0,
        bc0[:LH].reshape(1, LH), bc0[2 * LH:3 * LH].reshape(1, LH),
        bc0[3 * LH:].reshape(1, LH),
        wi1, wg1, wo1,
        bc1[:LH].reshape(1, LH), bc1[2 * LH:3 * LH].reshape(1, LH),
        bc1[3 * LH:].reshape(1, LH),
        vW1, vb1.reshape(1, H // 2), vW2, vb2.reshape(1, 1),
        tW1, tb1.reshape(1, H // 2), tW2, tb2.reshape(1, NT))
    return vol[:N], typ[:N]


def kernel(x, edge_index, temporal_seq, W1, b1, W2, b2, W3, b3, Wih0, Whh0,
           bih0, bhh0, Wih1, Whh1, bih1, bhh1, vW1, vb1, vW2, vb2, tW1, tb1,
           tW2, tb2):
    del temporal_seq, Whh0, Whh1
    return _run(x, edge_index, W1, b1, W2, b2, W3, b3, Wih0, bih0, bhh0,
                Wih1, bih1, bhh1, vW1, vb1, vW2, vb2, tW1, tb1, tW2, tb2)


# speedup vs baseline: 1.0744x; 1.0744x over previous
import functools

import jax
import jax.numpy as jnp
from jax import lax
from jax.experimental import pallas as pl
from jax.experimental.pallas import tpu as pltpu
from jax.experimental.pallas import tpu_sc as plsc

N = 10000
E = 320000
F_IN = 128
H = 64
LH = 32
NT = 10

N_TAB = 10112
NC = 2
NS = 16
NW = NC * NS
CH = 128
CPW = 80
E_PAD = NW * CPW * CH
NPT = N_TAB // NS
WD = 16

_f32 = jnp.float32

_mesh = plsc.VectorSubcoreMesh(core_axis_name="c", subcore_axis_name="s")
_sc_params = pltpu.CompilerParams(use_tc_tiling_on_sc=False)


@functools.partial(
    pl.kernel,
    out_type=jax.ShapeDtypeStruct((NC, N_TAB, WD), _f32),
    mesh=_mesh,
    scratch_types=[
        pltpu.VMEM((CPW, CH), jnp.int32),
        pltpu.VMEM((CH, WD), _f32),
        pltpu.VMEM_SHARED((N_TAB, WD), _f32),
        pltpu.SemaphoreType.DMA,
    ],
    compiler_params=_sc_params,
)
def _deg_kernel(dst_hbm, ones_hbm, zeros_hbm, out_hbm, dst_v, ones_v,
                acc_sh, sem):
    cid = lax.axis_index("c")
    sid = lax.axis_index("s")
    wid = sid * NC + cid
    pltpu.sync_copy(zeros_hbm, acc_sh.at[pl.ds(sid * NPT, NPT)])
    pltpu.sync_copy(ones_hbm, ones_v)
    pltpu.sync_copy(dst_hbm.at[wid], dst_v)
    plsc.subcore_barrier()

    def body(j, carry):
        pltpu.async_copy(ones_v, acc_sh.at[dst_v.at[j]], sem, add=True)

        @pl.when(j >= 8)
        def _():
            pltpu.make_async_copy(ones_v, acc_sh.at[dst_v.at[j - 8]],
                                  sem).wait()

        return carry

    lax.fori_loop(0, CPW, body, 0)
    for k in range(8):
        pltpu.make_async_copy(ones_v, acc_sh.at[dst_v.at[CPW - 8 + k]],
                              sem).wait()
    plsc.subcore_barrier()
    pltpu.sync_copy(acc_sh.at[pl.ds(sid * NPT, NPT)],
                    out_hbm.at[cid, pl.ds(sid * NPT, NPT)])


@functools.partial(
    pl.kernel,
    out_type=jax.ShapeDtypeStruct((NC, N_TAB, H), _f32),
    mesh=_mesh,
    scratch_types=[
        pltpu.VMEM((CPW, CH), jnp.int32),
        pltpu.VMEM((CPW, CH), jnp.int32),
        [pltpu.VMEM((CH, H), _f32)] * 4,
        pltpu.VMEM_SHARED((N_TAB, H), _f32),
        [pltpu.SemaphoreType.DMA] * 4,
        [pltpu.SemaphoreType.DMA] * 3,
    ],
    compiler_params=_sc_params,
)
def _scatter_kernel(g_hbm, src_hbm, dst_hbm, zeros_hbm, out_hbm, src_v, dst_v,
                    rows, acc_sh, sems, psems):
    cid = lax.axis_index("c")
    sid = lax.axis_index("s")
    wid = sid * NC + cid
    pltpu.async_copy(zeros_hbm, acc_sh.at[pl.ds(sid * NPT, NPT)], psems[0])
    pltpu.async_copy(src_hbm.at[wid], src_v, psems[1])
    pltpu.async_copy(dst_hbm.at[wid], dst_v, psems[2])
    pltpu.make_async_copy(src_hbm.at[wid], src_v, psems[1]).wait()

    NB = 4
    for b in range(NB):
        pltpu.async_copy(g_hbm.at[src_v.at[b]], rows[b], sems[b])

    pltpu.make_async_copy(dst_hbm.at[wid], dst_v, psems[2]).wait()
    pltpu.make_async_copy(zeros_hbm, acc_sh.at[pl.ds(sid * NPT, NPT)],
                          psems[0]).wait()
    plsc.subcore_barrier()

    def body(i, carry):
        j0 = NB * i
        for b in range(NB):
            j = j0 + b
            pltpu.make_async_copy(g_hbm.at[src_v.at[j]], rows[b],
                                  sems[b]).wait()
            pltpu.sync_copy(rows[b], acc_sh.at[dst_v.at[j]], add=True)

            @pl.when(j + NB < CPW)
            def _():
                pltpu.async_copy(g_hbm.at[src_v.at[j + NB]], rows[b], sems[b])

        return carry

    lax.fori_loop(0, CPW // NB, body, 0)
    plsc.subcore_barrier()
    pltpu.sync_copy(acc_sh.at[pl.ds(sid * NPT, NPT)],
                    out_hbm.at[cid, pl.ds(sid * NPT, NPT)])


def _prep_body(degacc, x, w1, dinv_o, g1_o):
    deg = degacc[0] + degacc[1]
    deg0 = deg[:, 0:1] + 1.0
    row = lax.broadcasted_iota(jnp.int32, (N_TAB, 1), 0)
    dinv = jnp.where(row < N, lax.rsqrt(deg0), 0.0)
    dinv_o[...] = dinv
    g1_o[...] = dinv * jnp.dot(x[...], w1[...], preferred_element_type=_f32)


_prep_call = pl.pallas_call(
    _prep_body,
    out_shape=(jax.ShapeDtypeStruct((N_TAB, 1), _f32),
               jax.ShapeDtypeStruct((N_TAB, H), _f32)),
)


def _mid_body(acc, g, dinv, b, w, gout):
    s = acc[0] + acc[1] + g[...]
    h = jnp.maximum(dinv[...] * s + b[...], 0.0)
    gout[...] = dinv[...] * jnp.dot(h, w[...], preferred_element_type=_f32)


_mid_call = pl.pallas_call(
    _mid_body,
    out_shape=jax.ShapeDtypeStruct((N_TAB, H), _f32),
)


def _epi_body(acc, g3, dinv, b3, wi0, wg0, wo0, bi0, bg0, bo0,
              wi1, wg1, wo1, bi1, bg1, bo1,
              vw1, vb1, vw2, vb2, tw1, tb1, tw2, tb2, vol_o, typ_o):
    s = acc[0] + acc[1] + g3[...]
    h = jnp.maximum(dinv[...] * s + b3[...], 0.0)

    def cell(hh, wi, wg, wo, bi, bg, bo):
        ig = jax.nn.sigmoid(jnp.dot(hh, wi[...], preferred_element_type=_f32)
                            + bi[...])
        gg = jnp.tanh(jnp.dot(hh, wg[...], preferred_element_type=_f32)
                      + bg[...])
        og = jax.nn.sigmoid(jnp.dot(hh, wo[...], preferred_element_type=_f32)
                            + bo[...])
        return og * jnp.tanh(ig * gg)

    hl = cell(h, wi0, wg0, wo0, bi0, bg0, bo0)
    hl = cell(hl, wi1, wg1, wo1, bi1, bg1, bo1)
    v = jnp.maximum(jnp.dot(hl, vw1[...], preferred_element_type=_f32)
                    + vb1[...], 0.0)
    vol_o[...] = jnp.dot(v, vw2[...], preferred_element_type=_f32) + vb2[...]
    t = jnp.maximum(jnp.dot(hl, tw1[...], preferred_element_type=_f32)
                    + tb1[...], 0.0)
    typ_o[...] = jnp.dot(t, tw2[...], preferred_element_type=_f32) + tb2[...]


_epi_call = pl.pallas_call(
    _epi_body,
    out_shape=(jax.ShapeDtypeStruct((N_TAB, 1), _f32),
               jax.ShapeDtypeStruct((N_TAB, NT), _f32)),
)


@jax.jit
def _run(x, edge_index, W1, b1, W2, b2, W3, b3, Wih0, bih0, bhh0,
         Wih1, bih1, bhh1, vW1, vb1, vW2, vb2, tW1, tb1, tW2, tb2):
    pad = E_PAD - E
    padrows = N + (jnp.arange(pad, dtype=jnp.int32) % (N_TAB - N))
    src_p = jnp.concatenate([edge_index[0], padrows]).reshape(NW, CPW, CH)
    dst_p = jnp.concatenate([edge_index[1], padrows]).reshape(NW, CPW, CH)
    x_pad = jnp.pad(x, ((0, N_TAB - N), (0, 0)))

    zeros16 = jnp.zeros((NPT, WD), _f32)
    ones16 = jnp.ones((CH, WD), _f32)
    zeros64 = jnp.zeros((NPT, H), _f32)

    degacc = _deg_kernel(dst_p, ones16, zeros16)
    dinv, g1 = _prep_call(degacc, x_pad, W1)
    acc1 = _scatter_kernel(g1, src_p, dst_p, zeros64)
    g2 = _mid_call(acc1, g1, dinv, b1.reshape(1, H), W2)
    acc2 = _scatter_kernel(g2, src_p, dst_p, zeros64)
    g3 = _mid_call(acc2, g2, dinv, b2.reshape(1, H), W3)
    acc3 = _scatter_kernel(g3, src_p, dst_p, zeros64)

    bc0 = bih0 + bhh0
    bc1 = bih1 + bhh1
    wi0, wg0, wo0 = Wih0[:LH].T, Wih0[2 * LH:3 * LH].T, Wih0[3 * LH:].T
    wi1, wg1, wo1 = Wih1[:LH].T, Wih1[2 * LH:3 * LH].T, Wih1[3 * LH:].T
    vol, typ = _epi_call(
        acc3, g3, dinv, b3.reshape(1, H),
        wi0, wg0, wo0,
        bc0[:LH].reshape(1, LH), bc0[2 * LH:3 * LH].reshape(1, LH),
        bc0[3 * LH:].reshape(1, LH),
        wi1, wg1, wo1,
        bc1[:LH].reshape(1, LH), bc1[2 * LH:3 * LH].reshape(1, LH),
        bc1[3 * LH:].reshape(1, LH),
        vW1, vb1.reshape(1, H // 2), vW2, vb2.reshape(1, 1),
        tW1, tb1.reshape(1, H // 2), tW2, tb2.reshape(1, NT))
    return vol[:N], typ[:N]


def kernel(x, edge_index, temporal_seq, W1, b1, W2, b2, W3, b3, Wih0, Whh0,
           bih0, bhh0, Wih1, Whh1, bih1, bhh1, vW1, vb1, vW2, vb2, tW1, tb1,
           tW2, tb2):
    del temporal_seq, Whh0, Whh1
    return _run(x, edge_index, W1, b1, W2, b2, W3, b3, Wih0, bih0, bhh0,
                Wih1, bih1, bhh1, vW1, vb1, vW2, vb2, tW1, tb1, tW2, tb2)


# speedup vs baseline: 1.0765x; 1.0020x over previous
import functools

import jax
import jax.numpy as jnp
from jax import lax
from jax.experimental import pallas as pl
from jax.experimental.pallas import tpu as pltpu
from jax.experimental.pallas import tpu_sc as plsc

N = 10000
E = 320000
F_IN = 128
H = 64
LH = 32
NT = 10

N_TAB = 10112
NC = 2
NS = 16
NW = NC * NS
CH = 128
CPW = 80
E_PAD = NW * CPW * CH
NPT = N_TAB // NS
WD = 16

_f32 = jnp.float32

_mesh = plsc.VectorSubcoreMesh(core_axis_name="c", subcore_axis_name="s")
_sc_params = pltpu.CompilerParams(use_tc_tiling_on_sc=False)


@functools.partial(
    pl.kernel,
    out_type=jax.ShapeDtypeStruct((NC, N_TAB, WD), _f32),
    mesh=_mesh,
    scratch_types=[
        pltpu.VMEM((CPW, CH), jnp.int32),
        pltpu.VMEM((CH, WD), _f32),
        pltpu.VMEM_SHARED((N_TAB, WD), _f32),
        pltpu.SemaphoreType.DMA,
        [pltpu.SemaphoreType.DMA] * 3,
    ],
    compiler_params=_sc_params,
)
def _deg_kernel(dst_hbm, ones_hbm, zeros_hbm, out_hbm, dst_v, ones_v,
                acc_sh, sem, psems):
    cid = lax.axis_index("c")
    sid = lax.axis_index("s")
    wid = sid * NC + cid
    pltpu.async_copy(zeros_hbm, acc_sh.at[pl.ds(sid * NPT, NPT)], psems[0])
    pltpu.async_copy(ones_hbm, ones_v, psems[1])
    pltpu.async_copy(dst_hbm.at[wid], dst_v, psems[2])
    pltpu.make_async_copy(ones_hbm, ones_v, psems[1]).wait()
    pltpu.make_async_copy(dst_hbm.at[wid], dst_v, psems[2]).wait()
    pltpu.make_async_copy(zeros_hbm, acc_sh.at[pl.ds(sid * NPT, NPT)],
                          psems[0]).wait()
    plsc.subcore_barrier()

    def body(j, carry):
        pltpu.async_copy(ones_v, acc_sh.at[dst_v.at[j]], sem, add=True)

        @pl.when(j >= 8)
        def _():
            pltpu.make_async_copy(ones_v, acc_sh.at[dst_v.at[j - 8]],
                                  sem).wait()

        return carry

    lax.fori_loop(0, CPW, body, 0)
    for k in range(8):
        pltpu.make_async_copy(ones_v, acc_sh.at[dst_v.at[CPW - 8 + k]],
                              sem).wait()
    plsc.subcore_barrier()
    pltpu.sync_copy(acc_sh.at[pl.ds(sid * NPT, NPT)],
                    out_hbm.at[cid, pl.ds(sid * NPT, NPT)])


@functools.partial(
    pl.kernel,
    out_type=jax.ShapeDtypeStruct((NC, N_TAB, H), _f32),
    mesh=_mesh,
    scratch_types=[
        pltpu.VMEM((CPW, CH), jnp.int32),
        pltpu.VMEM((CPW, CH), jnp.int32),
        [pltpu.VMEM((CH, H), _f32)] * 4,
        pltpu.VMEM_SHARED((N_TAB, H), _f32),
        [pltpu.SemaphoreType.DMA] * 4,
        [pltpu.SemaphoreType.DMA] * 3,
    ],
    compiler_params=_sc_params,
)
def _scatter_kernel(g_hbm, src_hbm, dst_hbm, zeros_hbm, out_hbm, src_v, dst_v,
                    rows, acc_sh, sems, psems):
    cid = lax.axis_index("c")
    sid = lax.axis_index("s")
    wid = sid * NC + cid
    pltpu.async_copy(zeros_hbm, acc_sh.at[pl.ds(sid * NPT, NPT)], psems[0])
    pltpu.async_copy(src_hbm.at[wid], src_v, psems[1])
    pltpu.async_copy(dst_hbm.at[wid], dst_v, psems[2])
    pltpu.make_async_copy(src_hbm.at[wid], src_v, psems[1]).wait()

    NB = 4
    for b in range(NB):
        pltpu.async_copy(g_hbm.at[src_v.at[b]], rows[b], sems[b])

    pltpu.make_async_copy(dst_hbm.at[wid], dst_v, psems[2]).wait()
    pltpu.make_async_copy(zeros_hbm, acc_sh.at[pl.ds(sid * NPT, NPT)],
                          psems[0]).wait()
    plsc.subcore_barrier()

    def body(i, carry):
        j0 = NB * i
        for b in range(NB):
            j = j0 + b
            pltpu.make_async_copy(g_hbm.at[src_v.at[j]], rows[b],
                                  sems[b]).wait()
            pltpu.sync_copy(rows[b], acc_sh.at[dst_v.at[j]], add=True)

            @pl.when(j + NB < CPW)
            def _():
                pltpu.async_copy(g_hbm.at[src_v.at[j + NB]], rows[b], sems[b])

        return carry

    lax.fori_loop(0, CPW // NB, body, 0)
    plsc.subcore_barrier()
    pltpu.sync_copy(acc_sh.at[pl.ds(sid * NPT, NPT)],
                    out_hbm.at[cid, pl.ds(sid * NPT, NPT)])


def _prep_body(degacc, x, w1, dinv_o, g1_o):
    deg = degacc[0] + degacc[1]
    deg0 = deg[:, 0:1] + 1.0
    row = lax.broadcasted_iota(jnp.int32, (N_TAB, 1), 0)
    dinv = jnp.where(row < N, lax.rsqrt(deg0), 0.0)
    dinv_o[...] = dinv
    g1_o[...] = dinv * jnp.dot(x[...], w1[...], preferred_element_type=_f32)


_prep_call = pl.pallas_call(
    _prep_body,
    out_shape=(jax.ShapeDtypeStruct((N_TAB, 1), _f32),
               jax.ShapeDtypeStruct((N_TAB, H), _f32)),
)


def _mid_body(acc, g, dinv, b, w, gout):
    s = acc[0] + acc[1] + g[...]
    h = jnp.maximum(dinv[...] * s + b[...], 0.0)
    gout[...] = dinv[...] * jnp.dot(h, w[...], preferred_element_type=_f32)


_mid_call = pl.pallas_call(
    _mid_body,
    out_shape=jax.ShapeDtypeStruct((N_TAB, H), _f32),
)


def _epi_body(acc, g3, dinv, b3, wi0, wg0, wo0, bi0, bg0, bo0,
              wi1, wg1, wo1, bi1, bg1, bo1,
              vw1, vb1, vw2, vb2, tw1, tb1, tw2, tb2, vol_o, typ_o):
    s = acc[0] + acc[1] + g3[...]
    h = jnp.maximum(dinv[...] * s + b3[...], 0.0)

    def cell(hh, wi, wg, wo, bi, bg, bo):
        ig = jax.nn.sigmoid(jnp.dot(hh, wi[...], preferred_element_type=_f32)
                            + bi[...])
        gg = jnp.tanh(jnp.dot(hh, wg[...], preferred_element_type=_f32)
                      + bg[...])
        og = jax.nn.sigmoid(jnp.dot(hh, wo[...], preferred_element_type=_f32)
                            + bo[...])
        return og * jnp.tanh(ig * gg)

    hl = cell(h, wi0, wg0, wo0, bi0, bg0, bo0)
    hl = cell(hl, wi1, wg1, wo1, bi1, bg1, bo1)
    v = jnp.maximum(jnp.dot(hl, vw1[...], preferred_element_type=_f32)
                    + vb1[...], 0.0)
    vol_o[...] = jnp.dot(v, vw2[...], preferred_element_type=_f32) + vb2[...]
    t = jnp.maximum(jnp.dot(hl, tw1[...], preferred_element_type=_f32)
                    + tb1[...], 0.0)
    typ_o[...] = jnp.dot(t, tw2[...], preferred_element_type=_f32) + tb2[...]


_epi_call = pl.pallas_call(
    _epi_body,
    out_shape=(jax.ShapeDtypeStruct((N_TAB, 1), _f32),
               jax.ShapeDtypeStruct((N_TAB, NT), _f32)),
)


@jax.jit
def _run(x, edge_index, W1, b1, W2, b2, W3, b3, Wih0, bih0, bhh0,
         Wih1, bih1, bhh1, vW1, vb1, vW2, vb2, tW1, tb1, tW2, tb2):
    pad = E_PAD - E
    padrows = N + (jnp.arange(pad, dtype=jnp.int32) % (N_TAB - N))
    src_p = jnp.concatenate([edge_index[0], padrows]).reshape(NW, CPW, CH)
    dst_p = jnp.concatenate([edge_index[1], padrows]).reshape(NW, CPW, CH)
    x_pad = jnp.pad(x, ((0, N_TAB - N), (0, 0)))

    zeros16 = jnp.zeros((NPT, WD), _f32)
    ones16 = jnp.ones((CH, WD), _f32)
    zeros64 = jnp.zeros((NPT, H), _f32)

    degacc = _deg_kernel(dst_p, ones16, zeros16)
    dinv, g1 = _prep_call(degacc, x_pad, W1)
    acc1 = _scatter_kernel(g1, src_p, dst_p, zeros64)
    g2 = _mid_call(acc1, g1, dinv, b1.reshape(1, H), W2)
    acc2 = _scatter_kernel(g2, src_p, dst_p, zeros64)
    g3 = _mid_call(acc2, g2, dinv, b2.reshape(1, H), W3)
    acc3 = _scatter_kernel(g3, src_p, dst_p, zeros64)

    bc0 = bih0 + bhh0
    bc1 = bih1 + bhh1
    wi0, wg0, wo0 = Wih0[:LH].T, Wih0[2 * LH:3 * LH].T, Wih0[3 * LH:].T
    wi1, wg1, wo1 = Wih1[:LH].T, Wih1[2 * LH:3 * LH].T, Wih1[3 * LH:].T
    vol, typ = _epi_call(
        acc3, g3, dinv, b3.reshape(1, H),
        wi0, wg0, wo0,
        bc0[:LH].reshape(1, LH), bc0[2 * LH:3 * LH].reshape(1, LH),
        bc0[3 * LH:].reshape(1, LH),
        wi1, wg1, wo1,
        bc1[:LH].reshape(1, LH), bc1[2 * LH:3 * LH].reshape(1, LH),
        bc1[3 * LH:].reshape(1, LH),
        vW1, vb1.reshape(1, H // 2), vW2, vb2.reshape(1, 1),
        tW1, tb1.reshape(1, H // 2), tW2, tb2.reshape(1, NT))
    return vol[:N], typ[:N]


def kernel(x, edge_index, temporal_seq, W1, b1, W2, b2, W3, b3, Wih0, Whh0,
           bih0, bhh0, Wih1, Whh1, bih1, bhh1, vW1, vb1, vW2, vb2, tW1, tb1,
           tW2, tb2):
    del temporal_seq, Whh0, Whh1
    return _run(x, edge_index, W1, b1, W2, b2, W3, b3, Wih0, bih0, bhh0,
                Wih1, bih1, bhh1, vW1, vb1, vW2, vb2, tW1, tb1, tW2, tb2)
